# Initial kernel scaffold; baseline (speedup 1.0000x reference)
import jax
import jax.numpy as jnp
from jax.experimental import pallas as pl


def kernel(x1, x2, edge_index, W, b):
    raise NotImplementedError("write your pallas kernel here")


# speedup vs baseline: 1.1355x; 1.1355x over previous
import functools

import jax
import jax.numpy as jnp
from jax.experimental import pallas as pl
from jax.experimental.pallas import tpu as pltpu

_BN = 400


def _linnorm_body(u0, u1, u2, u3, v0, v1, v2, v3, w_ref, b_ref, out1, out2):
    us = (u0[...], u1[...], u2[...], u3[...])
    vs = (v0[...], v1[...], v2[...], v3[...])
    coef = ((1.0, 0.0, 0.0, 0.0),
            (1.0, 1.0, 0.0, 0.0),
            (1.0, 2.0, 1.0, 0.0),
            (1.0, 3.0, 3.0, 1.0))
    for k in range(4):
        xa = coef[k][0] * us[0]
        xb = coef[k][0] * vs[0]
        for j in range(1, k + 1):
            xa = xa + coef[k][j] * us[j]
            sgn = -coef[k][j] if (j % 2) else coef[k][j]
            xb = xb + sgn * vs[j]
        wk = w_ref[k]
        bk = b_ref[k][None, :]
        oa = jnp.dot(xa, wk, preferred_element_type=jnp.float32) + bk
        ob = jnp.dot(xb, wk, preferred_element_type=jnp.float32) + bk
        na = jnp.maximum(jnp.sqrt(jnp.sum(oa * oa, axis=1, keepdims=True)), 1e-12)
        nb = jnp.maximum(jnp.sqrt(jnp.sum(ob * ob, axis=1, keepdims=True)), 1e-12)
        out1[:, k * 256:(k + 1) * 256] = oa / na
        out2[:, k * 256:(k + 1) * 256] = ob / nb


def _linnorm(us, vs, W, b):
    n = us[0].shape[0]
    grid = n // _BN
    blk = pl.BlockSpec((_BN, 256), lambda i: (i, 0))
    full_w = pl.BlockSpec((4, 256, 256), lambda i: (0, 0, 0))
    full_b = pl.BlockSpec((4, 256), lambda i: (0, 0))
    out_blk = pl.BlockSpec((_BN, 1024), lambda i: (i, 0))
    return pl.pallas_call(
        _linnorm_body,
        grid=(grid,),
        in_specs=[blk] * 8 + [full_w, full_b],
        out_specs=[out_blk, out_blk],
        out_shape=[jax.ShapeDtypeStruct((n, 1024), jnp.float32)] * 2,
    )(*us, *vs, W, b)


def kernel(x1, x2, edge_index, W, b):
    n = x1.shape[0]
    row = edge_index[0]
    col = edge_index[1]
    deg = jnp.zeros((n,), jnp.float32).at[row].add(1.0)
    dis = jnp.where(deg > 0, jax.lax.rsqrt(jnp.where(deg > 0, deg, 1.0)), 0.0)
    w_e = dis[row] * dis[col]

    us = [x1]
    vs = [x2]
    for _ in range(3):
        us.append(jnp.zeros((n, 256), jnp.float32).at[col].add(us[-1][row] * w_e[:, None]))
        vs.append(jnp.zeros((n, 256), jnp.float32).at[col].add(vs[-1][row] * w_e[:, None]))
    out1, out2 = _linnorm(us, vs, W, b)
    return out1, out2


# speedup vs baseline: 5.7651x; 5.0770x over previous
import functools

import jax
import jax.numpy as jnp
from jax import lax
from jax.experimental import pallas as pl
from jax.experimental.pallas import tpu as pltpu
from jax.experimental.pallas import tpu_sc as plsc

N = 10000
NP = 10240
R = NP // 16
FH = 128
E = 160000
CH = 64
EPT = 10112
NCHUNK = EPT // CH
EPAD = 16 * EPT
PAD_IDX = 10016
HB = 64

_BN = 400


def _rsqrt16(x):
    i = plsc.bitcast(x, jnp.int32)
    i = jnp.int32(0x5F3759DF) - (i >> 1)
    y = plsc.bitcast(i, jnp.float32)
    for _ in range(3):
        y = y * (1.5 - 0.5 * x * y * y)
    return jnp.where(x > 0.5, y, 0.0)


def _bcast16(ref, idx):
    return plsc.load_gather(ref, [jnp.full((16,), idx, jnp.int32)])


def _sc_body(rowp, colp, x1c, x2c,
             u1, u2, u3, v1, v2, v3, ya, yb,
             row_v, gb0, gb1, cb0, cb1, deg2, ident, degblk, disblk, dis_sl,

             sem0, sem1, semc0, semc1,
             acc, sdeg2, sdis):
    c = lax.axis_index("c")
    t = lax.axis_index("s")
    coff = c * NP
    zero16 = jnp.zeros((16,), jnp.float32)
    ones16 = jnp.full((16,), 1.0, jnp.float32)

    pltpu.sync_copy(rowp.at[t], row_v)

    def zero_deg(i, _):
        for k in range(8):
            deg2[i, pl.ds(k * 16, 16)] = zero16
        return 0
    lax.fori_loop(0, NP // 128, zero_deg, 0)

    for i in range(5):
        for k in range(8):
            degblk[i, pl.ds(k * 16, 16)] = zero16
    for k in range(5):
        ident[0, pl.ds(k * 16, 16)] = lax.iota(jnp.int32, 16) + k * 16

    pltpu.sync_copy(degblk, sdeg2.at[pl.ds(t * 5, 5)])
    plsc.subcore_barrier()

    def deg_body(j, _):
        for k in range(8):
            idx = row_v[j, pl.ds(k * 16, 16)]
            plsc.addupdate_scatter(deg2, [idx >> 7, idx & 127], ones16)
        return 0
    lax.fori_loop(0, NP // 128 - 1, deg_body, 0)

    pltpu.sync_copy(deg2, sdeg2.at[ident.at[0]], add=True)
    plsc.subcore_barrier()

    pltpu.sync_copy(sdeg2.at[pl.ds(t * 5, 5)], degblk)
    for i in range(5):
        for k in range(8):
            disblk[pl.ds(i * 128 + k * 16, 16)] = _rsqrt16(
                degblk[i, pl.ds(k * 16, 16)])

    pltpu.sync_copy(disblk, sdis.at[pl.ds(t * R, R)])
    plsc.subcore_barrier()

    def off_body(j, _):
        for k in range(8):
            sl = pl.ds(k * 16, 16)
            row_v[j, sl] = row_v[j, sl] + coff
        return 0
    lax.fori_loop(0, NP // 128 - 1, off_body, 0)

    def zero_gb0(i, _):
        for k in range(FH // 16):
            gb0[i, pl.ds(k * 16, 16)] = zero16
        return 0

    def scale_gb1(r, _):
        dv = _bcast16(dis_sl, r)
        for k in range(FH // 16):
            sl = pl.ds(k * 16, 16)
            gb1[r, sl] = gb1[r, sl] * dv
        return 0

    for xc, outs in ((x1c, (u1, u2, u3)), (x2c, (v1, v2, v3))):
        for blk in range(R // HB):
            lb = t * R + blk * HB
            g = coff + lb
            pltpu.sync_copy(xc.at[pl.ds(g, HB)], gb1)
            pltpu.sync_copy(sdis.at[pl.ds(lb, HB)], dis_sl)
            lax.fori_loop(0, HB, scale_gb1, 0)
            pltpu.sync_copy(gb1, ya.at[pl.ds(g, HB)])
        plsc.subcore_barrier()

        for hop in range(3):
            ysrc = (ya, yb, ya)[hop]
            ydst = (yb, ya, None)[hop]
            uout = outs[hop]

            lax.fori_loop(0, CH, zero_gb0, 0)
            for i in range(R // CH):
                pltpu.sync_copy(gb0, acc.at[pl.ds(t * R + i * CH, CH)])
            plsc.subcore_barrier()

            def ridx(j, h):
                return row_v.at[j, pl.ds(h * CH, CH)]

            pltpu.async_copy(colp.at[t, 0], cb0, semc0)
            pltpu.async_copy(ysrc.at[ridx(0, 0)], gb0, sem0)

            def pipe_body(j, _):
                jj = j // 2

                @pl.when(j % 2 == 0)
                def _():
                    pltpu.async_copy(colp.at[t, j + 1], cb1, semc1)
                    pltpu.async_copy(ysrc.at[ridx(jj, 1)], gb1, sem1)
                    pltpu.make_async_copy(ysrc.at[ridx(jj, 0)], gb0,
                                          sem0).wait()
                    pltpu.make_async_copy(colp.at[t, j], cb0, semc0).wait()
                    pltpu.sync_copy(gb0, acc.at[cb0], add=True)

                @pl.when(j % 2 == 1)
                def _():
                    pltpu.async_copy(colp.at[t, j + 1], cb0, semc0)
                    pltpu.async_copy(ysrc.at[ridx(jj + 1, 0)], gb0, sem0)
                    pltpu.make_async_copy(ysrc.at[ridx(jj, 1)], gb1,
                                          sem1).wait()
                    pltpu.make_async_copy(colp.at[t, j], cb1, semc1).wait()
                    pltpu.sync_copy(gb1, acc.at[cb1], add=True)
                return 0

            lax.fori_loop(0, NCHUNK - 1, pipe_body, 0)
            last = NCHUNK - 1
            pltpu.make_async_copy(ysrc.at[ridx(last // 2, 1)], gb1,
                                  sem1).wait()
            pltpu.make_async_copy(colp.at[t, last], cb1, semc1).wait()
            pltpu.sync_copy(gb1, acc.at[cb1], add=True)
            plsc.subcore_barrier()

            for blk in range(R // HB):
                lb = t * R + blk * HB
                g = coff + lb
                pltpu.sync_copy(acc.at[pl.ds(lb, HB)], gb1)
                pltpu.sync_copy(sdis.at[pl.ds(lb, HB)], dis_sl)
                lax.fori_loop(0, HB, scale_gb1, 0)
                pltpu.sync_copy(gb1, uout.at[pl.ds(g, HB)])
                if ydst is not None:
                    lax.fori_loop(0, HB, scale_gb1, 0)
                    pltpu.sync_copy(gb1, ydst.at[pl.ds(g, HB)])
            plsc.subcore_barrier()


@jax.jit
def _sc_propagate(rowp, colp, x1c, x2c):
    out = jax.ShapeDtypeStruct((2 * NP, FH), jnp.float32)
    mesh = plsc.VectorSubcoreMesh(core_axis_name="c", subcore_axis_name="s")
    f = pl.kernel(
        _sc_body,
        out_type=[out] * 8,
        mesh=mesh,
        scratch_types=[
            pltpu.VMEM((NP // 128 - 1, 128), jnp.int32),
            pltpu.VMEM((CH, FH), jnp.float32),
            pltpu.VMEM((HB, FH), jnp.float32),
            pltpu.VMEM((CH,), jnp.int32),
            pltpu.VMEM((CH,), jnp.int32),
            pltpu.VMEM((NP // 128, 128), jnp.float32),
            pltpu.VMEM((1, 80), jnp.int32),
            pltpu.VMEM((5, 128), jnp.float32),
            pltpu.VMEM((R,), jnp.float32),
            pltpu.VMEM((HB,), jnp.float32),
            pltpu.SemaphoreType.DMA,
            pltpu.SemaphoreType.DMA,
            pltpu.SemaphoreType.DMA,
            pltpu.SemaphoreType.DMA,
            pltpu.VMEM_SHARED((NP, FH), jnp.float32),
            pltpu.VMEM_SHARED((NP // 128, 128), jnp.float32),
            pltpu.VMEM_SHARED((NP,), jnp.float32),
        ],
        compiler_params=pltpu.CompilerParams(needs_layout_passes=False),
    )
    return f(rowp, colp, x1c, x2c)


def _linnorm_body(u0, u1, u2, u3, v0, v1, v2, v3, w_ref, b_ref, out1, out2):
    us = (u0[...], u1[...], u2[...], u3[...])
    vs = (v0[...], v1[...], v2[...], v3[...])
    coef = ((1.0, 0.0, 0.0, 0.0),
            (1.0, 1.0, 0.0, 0.0),
            (1.0, 2.0, 1.0, 0.0),
            (1.0, 3.0, 3.0, 1.0))
    for k in range(4):
        xa = coef[k][0] * us[0]
        xb = coef[k][0] * vs[0]
        for j in range(1, k + 1):
            xa = xa + coef[k][j] * us[j]
            sgn = -coef[k][j] if (j % 2) else coef[k][j]
            xb = xb + sgn * vs[j]
        wk = w_ref[k]
        bk = b_ref[k][None, :]
        oa = jnp.dot(xa, wk, preferred_element_type=jnp.float32) + bk
        ob = jnp.dot(xb, wk, preferred_element_type=jnp.float32) + bk
        na = jnp.maximum(jnp.sqrt(jnp.sum(oa * oa, axis=1, keepdims=True)), 1e-12)
        nb = jnp.maximum(jnp.sqrt(jnp.sum(ob * ob, axis=1, keepdims=True)), 1e-12)
        out1[:, k * 256:(k + 1) * 256] = oa / na
        out2[:, k * 256:(k + 1) * 256] = ob / nb


def _linnorm(us, vs, W, b):
    n = us[0].shape[0]
    grid = n // _BN
    blk = pl.BlockSpec((_BN, 256), lambda i: (i, 0))
    full_w = pl.BlockSpec((4, 256, 256), lambda i: (0, 0, 0))
    full_b = pl.BlockSpec((4, 256), lambda i: (0, 0))
    out_blk = pl.BlockSpec((_BN, 1024), lambda i: (i, 0))
    return pl.pallas_call(
        _linnorm_body,
        grid=(grid,),
        in_specs=[blk] * 8 + [full_w, full_b],
        out_specs=[out_blk, out_blk],
        out_shape=[jax.ShapeDtypeStruct((n, 1024), jnp.float32)] * 2,
    )(*us, *vs, W, b)


def kernel(x1, x2, edge_index, W, b):
    row = edge_index[0]
    col = edge_index[1]
    pad = jnp.full((EPAD - E,), PAD_IDX, jnp.int32)
    rowp = jnp.concatenate([row, pad]).reshape(16, EPT // 128, 128)
    colp = jnp.concatenate([col, pad]).reshape(16, NCHUNK, CH)

    def cat_halves(x):
        xc = jnp.zeros((2 * NP, FH), jnp.float32)
        xc = xc.at[:N].set(x[:, :FH])
        return xc.at[NP:NP + N].set(x[:, FH:])

    x1c = cat_halves(x1)
    x2c = cat_halves(x2)

    u1, u2, u3, v1, v2, v3, _, _ = _sc_propagate(rowp, colp, x1c, x2c)

    def uncat(u):
        return jnp.concatenate([u[:N], u[NP:NP + N]], axis=1)

    us = [x1, uncat(u1), uncat(u2), uncat(u3)]
    vs = [x2, uncat(v1), uncat(v2), uncat(v3)]
    out1, out2 = _linnorm(us, vs, W, b)
    return out1, out2


# speedup vs baseline: 5.8403x; 1.0131x over previous
import functools

import jax
import jax.numpy as jnp
from jax import lax
from jax.experimental import pallas as pl
from jax.experimental.pallas import tpu as pltpu
from jax.experimental.pallas import tpu_sc as plsc

N = 10000
NP = 10240
R = NP // 16
FH = 128
E = 160000
CH = 64
EPT = 10112
NCHUNK = EPT // CH
EPAD = 16 * EPT
PAD_IDX = 10016
HB = 64

_BN = 400


def _rsqrt16(x):
    i = plsc.bitcast(x, jnp.int32)
    i = jnp.int32(0x5F3759DF) - (i >> 1)
    y = plsc.bitcast(i, jnp.float32)
    for _ in range(3):
        y = y * (1.5 - 0.5 * x * y * y)
    return jnp.where(x > 0.5, y, 0.0)


def _bcast16(ref, idx):
    return plsc.load_gather(ref, [jnp.full((16,), idx, jnp.int32)])


def _sc_body(rowp, colp, x1c, x2c,
             u1, u2, u3, v1, v2, v3, ya, yb,
             row_v, gb0, gb1, cb0, cb1, deg2, ident, degblk, disblk, dis_sl,

             sem0, sem1, semc0, semc1,
             acc, sdeg2, sdis):
    c = lax.axis_index("c")
    t = lax.axis_index("s")
    coff = c * NP
    zero16 = jnp.zeros((16,), jnp.float32)
    ones16 = jnp.full((16,), 1.0, jnp.float32)

    pltpu.sync_copy(rowp.at[t], row_v)

    def zero_deg(i, _):
        for k in range(8):
            deg2[i, pl.ds(k * 16, 16)] = zero16
        return 0
    lax.fori_loop(0, NP // 128, zero_deg, 0)

    for i in range(5):
        for k in range(8):
            degblk[i, pl.ds(k * 16, 16)] = zero16
    for k in range(5):
        ident[0, pl.ds(k * 16, 16)] = lax.iota(jnp.int32, 16) + k * 16

    pltpu.sync_copy(degblk, sdeg2.at[pl.ds(t * 5, 5)])
    plsc.subcore_barrier()

    def deg_body(j, _):
        for k in range(8):
            idx = row_v[j, pl.ds(k * 16, 16)]
            plsc.addupdate_scatter(deg2, [idx >> 7, idx & 127], ones16)
        return 0
    lax.fori_loop(0, NP // 128 - 1, deg_body, 0)

    pltpu.sync_copy(deg2, sdeg2.at[ident.at[0]], add=True)
    plsc.subcore_barrier()

    pltpu.sync_copy(sdeg2.at[pl.ds(t * 5, 5)], degblk)
    for i in range(5):
        for k in range(8):
            disblk[pl.ds(i * 128 + k * 16, 16)] = _rsqrt16(
                degblk[i, pl.ds(k * 16, 16)])

    pltpu.sync_copy(disblk, sdis.at[pl.ds(t * R, R)])
    plsc.subcore_barrier()

    def off_body(j, _):
        for k in range(8):
            sl = pl.ds(k * 16, 16)
            row_v[j, sl] = row_v[j, sl] + coff
        return 0
    lax.fori_loop(0, NP // 128 - 1, off_body, 0)

    def zero_gb0(i, _):
        for k in range(FH // 16):
            gb0[i, pl.ds(k * 16, 16)] = zero16
        return 0

    def scale_gb1(r, _):
        dv = _bcast16(dis_sl, r)
        for k in range(FH // 16):
            sl = pl.ds(k * 16, 16)
            gb1[r, sl] = gb1[r, sl] * dv
        return 0

    for xc, outs in ((x1c, (u1, u2, u3)), (x2c, (v1, v2, v3))):
        for blk in range(R // HB):
            lb = t * R + blk * HB
            g = coff + lb
            pltpu.sync_copy(xc.at[pl.ds(lb, HB), pl.ds(c * FH, FH)], gb1)
            pltpu.sync_copy(sdis.at[pl.ds(lb, HB)], dis_sl)
            lax.fori_loop(0, HB, scale_gb1, 0)
            pltpu.sync_copy(gb1, ya.at[pl.ds(g, HB)])
        plsc.subcore_barrier()

        for hop in range(3):
            ysrc = (ya, yb, ya)[hop]
            ydst = (yb, ya, None)[hop]
            uout = outs[hop]

            lax.fori_loop(0, CH, zero_gb0, 0)
            for i in range(R // CH):
                pltpu.sync_copy(gb0, acc.at[pl.ds(t * R + i * CH, CH)])
            plsc.subcore_barrier()

            def ridx(j, h):
                return row_v.at[j, pl.ds(h * CH, CH)]

            pltpu.async_copy(colp.at[t, 0], cb0, semc0)
            pltpu.async_copy(ysrc.at[ridx(0, 0)], gb0, sem0)

            def pipe_body(j, _):
                jj = j // 2

                @pl.when(j % 2 == 0)
                def _():
                    pltpu.async_copy(colp.at[t, j + 1], cb1, semc1)
                    pltpu.async_copy(ysrc.at[ridx(jj, 1)], gb1, sem1)
                    pltpu.make_async_copy(ysrc.at[ridx(jj, 0)], gb0,
                                          sem0).wait()
                    pltpu.make_async_copy(colp.at[t, j], cb0, semc0).wait()
                    pltpu.sync_copy(gb0, acc.at[cb0], add=True)

                @pl.when(j % 2 == 1)
                def _():
                    pltpu.async_copy(colp.at[t, j + 1], cb0, semc0)
                    pltpu.async_copy(ysrc.at[ridx(jj + 1, 0)], gb0, sem0)
                    pltpu.make_async_copy(ysrc.at[ridx(jj, 1)], gb1,
                                          sem1).wait()
                    pltpu.make_async_copy(colp.at[t, j], cb1, semc1).wait()
                    pltpu.sync_copy(gb1, acc.at[cb1], add=True)
                return 0

            lax.fori_loop(0, NCHUNK - 1, pipe_body, 0)
            last = NCHUNK - 1
            pltpu.make_async_copy(ysrc.at[ridx(last // 2, 1)], gb1,
                                  sem1).wait()
            pltpu.make_async_copy(colp.at[t, last], cb1, semc1).wait()
            pltpu.sync_copy(gb1, acc.at[cb1], add=True)
            plsc.subcore_barrier()

            for blk in range(R // HB):
                lb = t * R + blk * HB
                g = coff + lb
                pltpu.sync_copy(acc.at[pl.ds(lb, HB)], gb1)
                pltpu.sync_copy(sdis.at[pl.ds(lb, HB)], dis_sl)
                lax.fori_loop(0, HB, scale_gb1, 0)
                pltpu.sync_copy(
                    gb1, uout.at[pl.ds(lb, HB), pl.ds(c * FH, FH)])
                if ydst is not None:
                    lax.fori_loop(0, HB, scale_gb1, 0)
                    pltpu.sync_copy(gb1, ydst.at[pl.ds(g, HB)])
            plsc.subcore_barrier()


@jax.jit
def _sc_propagate(rowp, colp, x1c, x2c):
    uv = jax.ShapeDtypeStruct((NP, 2 * FH), jnp.float32)
    yy = jax.ShapeDtypeStruct((2 * NP, FH), jnp.float32)
    mesh = plsc.VectorSubcoreMesh(core_axis_name="c", subcore_axis_name="s")
    f = pl.kernel(
        _sc_body,
        out_type=[uv] * 6 + [yy] * 2,
        mesh=mesh,
        scratch_types=[
            pltpu.VMEM((NP // 128 - 1, 128), jnp.int32),
            pltpu.VMEM((CH, FH), jnp.float32),
            pltpu.VMEM((HB, FH), jnp.float32),
            pltpu.VMEM((CH,), jnp.int32),
            pltpu.VMEM((CH,), jnp.int32),
            pltpu.VMEM((NP // 128, 128), jnp.float32),
            pltpu.VMEM((1, 80), jnp.int32),
            pltpu.VMEM((5, 128), jnp.float32),
            pltpu.VMEM((R,), jnp.float32),
            pltpu.VMEM((HB,), jnp.float32),
            pltpu.SemaphoreType.DMA,
            pltpu.SemaphoreType.DMA,
            pltpu.SemaphoreType.DMA,
            pltpu.SemaphoreType.DMA,
            pltpu.VMEM_SHARED((NP, FH), jnp.float32),
            pltpu.VMEM_SHARED((NP // 128, 128), jnp.float32),
            pltpu.VMEM_SHARED((NP,), jnp.float32),
        ],
        compiler_params=pltpu.CompilerParams(needs_layout_passes=False),
    )
    return f(rowp, colp, x1c, x2c)


def _linnorm_body(u0, u1, u2, u3, v0, v1, v2, v3, w_ref, b_ref, out1, out2):
    us = (u0[...], u1[...], u2[...], u3[...])
    vs = (v0[...], v1[...], v2[...], v3[...])
    coef = ((1.0, 0.0, 0.0, 0.0),
            (1.0, 1.0, 0.0, 0.0),
            (1.0, 2.0, 1.0, 0.0),
            (1.0, 3.0, 3.0, 1.0))
    for k in range(4):
        xa = coef[k][0] * us[0]
        xb = coef[k][0] * vs[0]
        for j in range(1, k + 1):
            xa = xa + coef[k][j] * us[j]
            sgn = -coef[k][j] if (j % 2) else coef[k][j]
            xb = xb + sgn * vs[j]
        wk = w_ref[k]
        bk = b_ref[k][None, :]
        oa = jnp.dot(xa, wk, preferred_element_type=jnp.float32) + bk
        ob = jnp.dot(xb, wk, preferred_element_type=jnp.float32) + bk
        na = jnp.maximum(jnp.sqrt(jnp.sum(oa * oa, axis=1, keepdims=True)), 1e-12)
        nb = jnp.maximum(jnp.sqrt(jnp.sum(ob * ob, axis=1, keepdims=True)), 1e-12)
        out1[:, k * 256:(k + 1) * 256] = oa / na
        out2[:, k * 256:(k + 1) * 256] = ob / nb


def _linnorm(us, vs, W, b):
    n = us[0].shape[0]
    grid = n // _BN
    blk = pl.BlockSpec((_BN, 256), lambda i: (i, 0))
    full_w = pl.BlockSpec((4, 256, 256), lambda i: (0, 0, 0))
    full_b = pl.BlockSpec((4, 256), lambda i: (0, 0))
    out_blk = pl.BlockSpec((_BN, 1024), lambda i: (i, 0))
    return pl.pallas_call(
        _linnorm_body,
        grid=(grid,),
        in_specs=[blk] * 8 + [full_w, full_b],
        out_specs=[out_blk, out_blk],
        out_shape=[jax.ShapeDtypeStruct((n, 1024), jnp.float32)] * 2,
    )(*us, *vs, W, b)


def kernel(x1, x2, edge_index, W, b):
    row = edge_index[0]
    col = edge_index[1]
    pad = jnp.full((EPAD - E,), PAD_IDX, jnp.int32)
    rowp = jnp.concatenate([row, pad]).reshape(16, EPT // 128, 128)
    colp = jnp.concatenate([col, pad]).reshape(16, NCHUNK, CH)

    def padx(x):
        return jnp.zeros((NP, 2 * FH), jnp.float32).at[:N].set(x)

    u1, u2, u3, v1, v2, v3, _, _ = _sc_propagate(
        rowp, colp, padx(x1), padx(x2))

    us = [x1, u1, u2, u3]
    vs = [x2, v1, v2, v3]
    out1, out2 = _linnorm(us, vs, W, b)
    return out1, out2
